# Initial kernel scaffold; baseline (speedup 1.0000x reference)
import jax
import jax.numpy as jnp
from jax.experimental import pallas as pl


def kernel(y_h, y, supervise):
    raise NotImplementedError("write your pallas kernel here")


# speedup vs baseline: 1.5583x; 1.5583x over previous
import functools

import jax
import jax.numpy as jnp
from jax.experimental import pallas as pl

ALPHA = 0.5
N_CLASSES = 1000
BATCH = 16384
BLOCK_ROWS = 256
GRID = BATCH // BLOCK_ROWS


def _loss_kernel(yh_ref, y_ref, m_ref, out_ref):
    i = pl.program_id(0)

    yh = yh_ref[...]
    yv = y_ref[0]

    row_max = jnp.max(yh, axis=1, keepdims=True)
    sumexp = jnp.sum(jnp.exp(yh - row_max), axis=1, keepdims=True)
    lse = jnp.log(sumexp) + row_max

    classes = jax.lax.broadcasted_iota(jnp.int32, (BLOCK_ROWS, N_CLASSES), 1)
    onehot = (classes == yv).astype(jnp.float32)

    picked = jnp.sum(onehot * yh, axis=1, keepdims=True)
    hard_sum = jnp.sum(lse - picked)

    td = jnp.dot(onehot.astype(jnp.bfloat16), m_ref[...],
                 preferred_element_type=jnp.float32)
    logp = yh - lse
    soft_sum = -jnp.sum(td * logp)

    contrib = (ALPHA * hard_sum + (1.0 - ALPHA) * soft_sum) / BATCH

    @pl.when(i == 0)
    def _():
        out_ref[...] = jnp.zeros_like(out_ref)

    out_ref[...] += contrib.reshape(1, 1)


@jax.jit
def kernel(y_h, y, supervise):
    m = supervise.T.astype(jnp.bfloat16)
    y2 = y.reshape(GRID, BLOCK_ROWS, 1)

    out = pl.pallas_call(
        _loss_kernel,
        grid=(GRID,),
        in_specs=[
            pl.BlockSpec((BLOCK_ROWS, N_CLASSES), lambda i: (i, 0)),
            pl.BlockSpec((1, BLOCK_ROWS, 1), lambda i: (i, 0, 0)),
            pl.BlockSpec((N_CLASSES, N_CLASSES), lambda i: (0, 0)),
        ],
        out_specs=pl.BlockSpec((1, 1), lambda i: (0, 0)),
        out_shape=jax.ShapeDtypeStruct((1, 1), jnp.float32),
    )(y_h, y2, m)
    return out[0, 0]
